# baseline (speedup 1.0000x reference)
import functools

import jax
import jax.numpy as jnp
from jax import lax
from jax.experimental import pallas as pl
from jax.experimental.pallas import tpu as pltpu
from jax.experimental.pallas import tpu_sc as plsc

N = 320000
FEAT_DIM = 128
HIDDEN = 512
LATENT = 128
NUM_SEGMENTS = 10000

NC = 2
NS = 16
NW = NC * NS
ROWS_PER_W = N // NW
CHUNK = 400
NCHUNK = ROWS_PER_W // CHUNK

ROW_TILE = 800
SEG_TILE = 1000


def _mlp_body(x_ref, w1_ref, b1_ref, w2_ref, b2_ref, o_ref):
    h = jnp.maximum(
        jnp.dot(x_ref[...], w1_ref[...], preferred_element_type=jnp.float32)
        + b1_ref[...], 0.0)
    e = jnp.maximum(
        jnp.dot(h.astype(jnp.bfloat16), w2_ref[...],
                preferred_element_type=jnp.float32)
        + b2_ref[...], 0.0)
    o_ref[...] = e.astype(jnp.bfloat16)


def _mlp(feat, W1, b1, W2, b2):
    grid = (N // ROW_TILE,)
    return pl.pallas_call(
        _mlp_body,
        grid=grid,
        in_specs=[
            pl.BlockSpec((ROW_TILE, FEAT_DIM), lambda i: (i, 0)),
            pl.BlockSpec((FEAT_DIM, HIDDEN), lambda i: (0, 0)),
            pl.BlockSpec((1, HIDDEN), lambda i: (0, 0)),
            pl.BlockSpec((HIDDEN, LATENT), lambda i: (0, 0)),
            pl.BlockSpec((1, LATENT), lambda i: (0, 0)),
        ],
        out_specs=pl.BlockSpec((ROW_TILE, LATENT), lambda i: (i, 0)),
        out_shape=jax.ShapeDtypeStruct((N, LATENT), jnp.bfloat16),
    )(feat, W1, b1, W2, b2)


def _segmax_body(emb_hbm, idx_hbm, direct_hbm, partials_hbm, pids_hbm,
                 idx_v, buf_v, stage_v, zrow_v, pid_v):
    w = lax.axis_index("s") * NC + lax.axis_index("c")
    base = w * ROWS_PER_W

    pltpu.sync_copy(idx_hbm.at[pl.ds(base, ROWS_PER_W)],
                    idx_v.at[pl.ds(0, ROWS_PER_W)])

    def idx_at(r):
        return idx_v[pl.ds(r, 16)][0]

    for j in range(LATENT // 32):
        zrow_v[pl.ds(j * 32, 32)] = jnp.zeros((32,), jnp.bfloat16)

    first_id = idx_at(0)

    def flush(cur, m, first_open):
        for j in range(LATENT // 32):
            stage_v[pl.ds(j * 32, 32)] = m[j]

        def to_partial():
            pltpu.sync_copy(stage_v,
                            partials_hbm.at[pl.ds(2 * w * LATENT, LATENT)])

        def to_direct():
            pltpu.sync_copy(stage_v,
                            direct_hbm.at[pl.ds(cur * LATENT, LATENT)])

        lax.cond(first_open == 1, to_partial, to_direct)

    def zero_gap(lo, hi):
        def body(g, _):
            pltpu.sync_copy(zrow_v, direct_hbm.at[pl.ds(g * LATENT, LATENT)])
            return 0
        lax.fori_loop(lo, hi, body, 0)

    def chunk_body(c, carry):
        pltpu.sync_copy(
            emb_hbm.at[pl.ds((base + c * CHUNK) * LATENT, CHUNK * LATENT)],
            buf_v)

        def row_body(r, carry):
            cur, first_open, m = carry
            s = idx_at(c * CHUNK + r)
            v = tuple(buf_v[pl.ds(r * LATENT + j * 32, 32)]
                      for j in range(LATENT // 32))

            changed = s != cur

            def on_change(_):
                flush(cur, m, first_open)
                zero_gap(cur + 1, s)
                return 0

            lax.cond(changed, on_change, lambda _: 0, 0)
            m_new = tuple(
                jnp.where(changed, v[j], jnp.maximum(m[j], v[j]))
                for j in range(LATENT // 32))
            return (jnp.where(changed, s, cur),
                    jnp.where(changed, jnp.int32(0), first_open),
                    m_new)

        return lax.fori_loop(0, CHUNK, row_body, carry)

    zeros32 = jnp.zeros((32,), jnp.bfloat16)
    init = (first_id, jnp.int32(1),
            tuple(zeros32 for _ in range(LATENT // 32)))
    cur, first_open, m = lax.fori_loop(0, NCHUNK, chunk_body, init)

    for j in range(LATENT // 32):
        stage_v[pl.ds(j * 32, 32)] = m[j]
    pltpu.sync_copy(stage_v, partials_hbm.at[pl.ds((2 * w + 1) * LATENT,
                                                   LATENT)])

    def also_first():
        pltpu.sync_copy(stage_v, partials_hbm.at[pl.ds(2 * w * LATENT,
                                                       LATENT)])

    def nothing():
        pass

    lax.cond(first_open == 1, also_first, nothing)

    lane = lax.broadcasted_iota(jnp.int32, (16,), 0)
    pid_v[...] = jnp.where(lane == 0, first_id,
                           jnp.where(lane == 1, cur, 0))
    pltpu.sync_copy(pid_v, pids_hbm.at[pl.ds(w * 16, 16)])


def _segmax(emb, idx):
    mesh = plsc.VectorSubcoreMesh(core_axis_name="c", subcore_axis_name="s")
    f = pl.kernel(
        _segmax_body,
        out_type=(
            jax.ShapeDtypeStruct((NUM_SEGMENTS * LATENT,), jnp.bfloat16),
            jax.ShapeDtypeStruct((2 * NW * LATENT,), jnp.bfloat16),
            jax.ShapeDtypeStruct((NW * 16,), jnp.int32),
        ),
        mesh=mesh,
        compiler_params=pltpu.CompilerParams(use_tc_tiling_on_sc=False),
        scratch_types=[
            pltpu.VMEM((ROWS_PER_W + 16,), jnp.int32),
            pltpu.VMEM((CHUNK * LATENT,), jnp.bfloat16),
            pltpu.VMEM((LATENT,), jnp.bfloat16),
            pltpu.VMEM((LATENT,), jnp.bfloat16),
            pltpu.VMEM((16,), jnp.int32),
        ],
    )
    return f(emb.reshape(-1), idx)


def _final_body(d_ref, p_ref, pid_ref, w3_ref, b3_ref, o_ref):
    i = pl.program_id(0)
    sid = lax.broadcasted_iota(jnp.int32, (SEG_TILE, 1), 0) + i * SEG_TILE

    clear = jnp.zeros((SEG_TILE, 1), jnp.bool_)
    for w in range(NW + 1):
        lo = jnp.int32(-1) if w == 0 else pid_ref[16 * (w - 1) + 1]
        hi = jnp.int32(NUM_SEGMENTS) if w == NW else pid_ref[16 * w]
        clear = jnp.logical_or(clear, jnp.logical_and(sid > lo, sid < hi))
    for k in range(2 * NW):
        pid = pid_ref[16 * (k // 2) + (k % 2)]
        clear = jnp.logical_or(clear, sid == pid)

    val = jnp.where(clear, jnp.bfloat16(0.0), d_ref[...])
    for k in range(2 * NW):
        pid = pid_ref[16 * (k // 2) + (k % 2)]
        prow = p_ref[k:k + 1, :]
        val = jnp.where(sid == pid, jnp.maximum(val, prow), val)

    o_ref[...] = (jnp.dot(val, w3_ref[...], preferred_element_type=jnp.float32)
                  + b3_ref[...])


def _final(direct, partials, pids, W3, b3):
    grid = (NUM_SEGMENTS // SEG_TILE,)
    return pl.pallas_call(
        _final_body,
        grid=grid,
        in_specs=[
            pl.BlockSpec((SEG_TILE, LATENT), lambda i: (i, 0)),
            pl.BlockSpec((2 * NW, LATENT), lambda i: (0, 0)),
            pl.BlockSpec(memory_space=pltpu.SMEM),
            pl.BlockSpec((LATENT, LATENT), lambda i: (0, 0)),
            pl.BlockSpec((1, LATENT), lambda i: (0, 0)),
        ],
        out_specs=pl.BlockSpec((SEG_TILE, LATENT), lambda i: (i, 0)),
        out_shape=jax.ShapeDtypeStruct((NUM_SEGMENTS, LATENT), jnp.float32),
    )(direct, partials, pids, W3, b3)


def kernel(feat, traj_inbatch_index, W1, b1, W2, b2, W3, b3):
    idx = traj_inbatch_index.astype(jnp.int32)
    emb = _mlp(feat.astype(jnp.bfloat16), W1.astype(jnp.bfloat16),
               b1.reshape(1, HIDDEN), W2.astype(jnp.bfloat16),
               b2.reshape(1, LATENT))
    direct, partials, pids = _segmax(emb, idx)
    return _final(direct.reshape(NUM_SEGMENTS, LATENT),
                  partials.reshape(2 * NW, LATENT), pids,
                  W3.astype(jnp.bfloat16), b3.reshape(1, LATENT))


# speedup vs baseline: 1.4469x; 1.4469x over previous
import jax
import jax.numpy as jnp
from jax import lax
from jax.experimental import pallas as pl
from jax.experimental.pallas import tpu as pltpu
from jax.experimental.pallas import tpu_sc as plsc

N = 320000
FEAT_DIM = 128
HIDDEN = 512
LATENT = 128
NUM_SEGMENTS = 10000
NJ = LATENT // 16

NC = 2
NS = 16
NW = NC * NS
ROWS_PER_W = N // NW
CHUNK = 400
NCHUNK = ROWS_PER_W // CHUNK
NBLK = CHUNK // 16
RING = 8

ROW_TILE = 800
SEG_TILE = 1000


def _mlp_body(x_ref, w1_ref, b1_ref, w2_ref, b2_ref, o_ref):
    h = jnp.maximum(
        jnp.dot(x_ref[...], w1_ref[...], preferred_element_type=jnp.float32)
        + b1_ref[...], 0.0)
    e = jnp.maximum(
        jnp.dot(h.astype(jnp.bfloat16), w2_ref[...],
                preferred_element_type=jnp.float32)
        + b2_ref[...], 0.0)
    o_ref[...] = e


def _mlp(feat, W1, b1, W2, b2):
    grid = (N // ROW_TILE,)
    return pl.pallas_call(
        _mlp_body,
        grid=grid,
        in_specs=[
            pl.BlockSpec((ROW_TILE, FEAT_DIM), lambda i: (i, 0)),
            pl.BlockSpec((FEAT_DIM, HIDDEN), lambda i: (0, 0)),
            pl.BlockSpec((1, HIDDEN), lambda i: (0, 0)),
            pl.BlockSpec((HIDDEN, LATENT), lambda i: (0, 0)),
            pl.BlockSpec((1, LATENT), lambda i: (0, 0)),
        ],
        out_specs=pl.BlockSpec((ROW_TILE, LATENT), lambda i: (i, 0)),
        out_shape=jax.ShapeDtypeStruct((N, LATENT), jnp.float32),
    )(feat, W1, b1, W2, b2)


def _segmax_body(emb_hbm, idx_hbm, direct_hbm, partials_hbm, pids_hbm,
                 idx_v, buf_v, mbuf_v, stage_v, zrow_v, pid_v,
                 flush_sem, chunk_sem):
    w = lax.axis_index("s") * NC + lax.axis_index("c")
    base = w * ROWS_PER_W
    zero16i = jnp.zeros((16,), jnp.int32)
    zero16f = jnp.zeros((16,), jnp.float32)

    idx_v[pl.ds(0, 16)] = zero16i - 1
    pltpu.sync_copy(idx_hbm.at[pl.ds(base, ROWS_PER_W)],
                    idx_v.at[pl.ds(16, ROWS_PER_W)])

    def idx_at(r):
        return idx_v[pl.ds(16 + r, 16)][0]

    for j in range(NJ):
        zrow_v[pl.ds(j * 16, 16)] = zero16f
        mbuf_v[pl.ds(j * 16, 16)] = zero16f

    def chunk_src(c):
        return emb_hbm.at[pl.ds((base + c * CHUNK) * LATENT, CHUNK * LATENT)]

    def chunk_dst(c):
        return buf_v.at[pl.ds(lax.rem(c, 2) * (CHUNK * LATENT),
                              CHUNK * LATENT)]

    def start_chunk(c):
        pltpu.async_copy(chunk_src(c), chunk_dst(c),
                         chunk_sem.at[lax.rem(c, 2)])

    def wait_chunk(c):
        pltpu.make_async_copy(chunk_src(c), chunk_dst(c),
                              chunk_sem.at[lax.rem(c, 2)]).wait()

    def stage_slot(slot):
        return stage_v.at[pl.ds(slot * LATENT, LATENT)]

    def flush(cur, m, first_open, k):
        slot = lax.rem(k, RING)

        def wait_slot():
            pltpu.make_async_copy(stage_slot(slot),
                                  direct_hbm.at[pl.ds(0, LATENT)],
                                  flush_sem.at[slot]).wait()

        lax.cond(jnp.logical_and(first_open == 0, k >= RING),
                 wait_slot, lambda: None)
        for j in range(NJ):
            stage_v[pl.ds(slot * LATENT + j * 16, 16)] = m[j]

        def to_partial():
            pltpu.sync_copy(stage_slot(slot),
                            partials_hbm.at[pl.ds(2 * w * LATENT, LATENT)])

        def to_direct():
            pltpu.async_copy(stage_slot(slot),
                             direct_hbm.at[pl.ds(cur * LATENT, LATENT)],
                             flush_sem.at[slot])

        lax.cond(first_open == 1, to_partial, to_direct)
        return jnp.where(first_open == 1, k, k + 1)

    def zero_gap(lo, hi):
        def body(g, _):
            pltpu.sync_copy(zrow_v, direct_hbm.at[pl.ds(g * LATENT, LATENT)])
            return 0
        lax.fori_loop(lo, hi, body, 0)

    def load_m():
        return tuple(mbuf_v[pl.ds(j * 16, 16)] for j in range(NJ))

    def store_m(m):
        for j in range(NJ):
            mbuf_v[pl.ds(j * 16, 16)] = m[j]

    def row_vals(boff, r):
        return tuple(buf_v[pl.ds(boff + r * LATENT + j * 16, 16)]
                     for j in range(NJ))

    def block_body(b, carry):
        c, cur, first_open, k = carry
        g = b * 16
        boff = (lax.rem(c, 2) * CHUNK + (g - c * CHUNK)) * LATENT
        a_vec = idx_v[pl.ds(16 + g, 16)]
        p_vec = idx_v[pl.ds(15 + g, 16)]
        nb = plsc.all_reduce_population_count(a_vec != p_vec)[0]

        def fast():
            m = load_m()
            rows = [row_vals(boff, r) for r in range(16)]
            while len(rows) > 1:
                rows = [tuple(jnp.maximum(x[j], y[j]) for j in range(NJ))
                        for x, y in zip(rows[::2], rows[1::2])]
            store_m(tuple(jnp.maximum(m[j], rows[0][j]) for j in range(NJ)))
            return cur, first_open, k

        def slow():
            def row_body(r, rcarry):
                rcur, ropen, rk = rcarry
                s = idx_at(g + r)
                v = row_vals(boff, r)
                changed = s != rcur
                m = load_m()

                def on_change(_):
                    nk = flush(rcur, m, ropen, rk)
                    zero_gap(rcur + 1, s)
                    return nk

                nk = lax.cond(changed, on_change, lambda _: rk, 0)
                store_m(tuple(
                    jnp.where(changed, v[j], jnp.maximum(m[j], v[j]))
                    for j in range(NJ)))
                return (jnp.where(changed, s, rcur),
                        jnp.where(changed, jnp.int32(0), ropen),
                        nk)

            return lax.fori_loop(0, 16, row_body, (cur, first_open, k))

        cur2, open2, k2 = lax.cond(nb == 0, fast, slow)
        return c, cur2, open2, k2

    def chunk_body(c, carry):
        def prefetch():
            start_chunk(c + 1)
        lax.cond(c < NCHUNK - 1, prefetch, lambda: None)
        wait_chunk(c)
        cur, first_open, k = carry
        _, cur, first_open, k = lax.fori_loop(
            c * NBLK, (c + 1) * NBLK, block_body, (c, cur, first_open, k))
        return cur, first_open, k

    start_chunk(0)
    init = (idx_at(0), jnp.int32(1), jnp.int32(0))
    cur, first_open, k = lax.fori_loop(0, NCHUNK, chunk_body, init)

    for s in range(RING):
        def drain():
            pltpu.make_async_copy(stage_slot(s),
                                  direct_hbm.at[pl.ds(0, LATENT)],
                                  flush_sem.at[s]).wait()
        lax.cond(k > s, drain, lambda: None)

    m = load_m()
    for j in range(NJ):
        stage_v[pl.ds(j * 16, 16)] = m[j]
    pltpu.sync_copy(stage_v.at[pl.ds(0, LATENT)],
                    partials_hbm.at[pl.ds((2 * w + 1) * LATENT, LATENT)])

    def also_first():
        pltpu.sync_copy(stage_v.at[pl.ds(0, LATENT)],
                        partials_hbm.at[pl.ds(2 * w * LATENT, LATENT)])

    lax.cond(first_open == 1, also_first, lambda: None)

    lane = lax.broadcasted_iota(jnp.int32, (16,), 0)
    pid_v[...] = jnp.where(lane == 0, idx_at(0),
                           jnp.where(lane == 1, cur, 0))
    pltpu.sync_copy(pid_v, pids_hbm.at[pl.ds(w * 16, 16)])


def _segmax(emb, idx):
    mesh = plsc.VectorSubcoreMesh(core_axis_name="c", subcore_axis_name="s")
    f = pl.kernel(
        _segmax_body,
        out_type=(
            jax.ShapeDtypeStruct((NUM_SEGMENTS * LATENT,), jnp.float32),
            jax.ShapeDtypeStruct((2 * NW * LATENT,), jnp.float32),
            jax.ShapeDtypeStruct((NW * 16,), jnp.int32),
        ),
        mesh=mesh,
        compiler_params=pltpu.CompilerParams(use_tc_tiling_on_sc=False,
                                             needs_layout_passes=False),
        scratch_types=[
            pltpu.VMEM((32 + ROWS_PER_W + 16,), jnp.int32),
            pltpu.VMEM((2 * CHUNK * LATENT,), jnp.float32),
            pltpu.VMEM((LATENT,), jnp.float32),
            pltpu.VMEM((RING * LATENT,), jnp.float32),
            pltpu.VMEM((LATENT,), jnp.float32),
            pltpu.VMEM((16,), jnp.int32),
            pltpu.SemaphoreType.DMA((RING,)),
            pltpu.SemaphoreType.DMA((2,)),
        ],
    )
    return f(emb.reshape(-1), idx)


def _final_body(d_ref, p_ref, pid_ref, w3_ref, b3_ref, o_ref):
    i = pl.program_id(0)
    sid = lax.broadcasted_iota(jnp.int32, (SEG_TILE, 1), 0) + i * SEG_TILE

    clear = jnp.zeros((SEG_TILE, 1), jnp.bool_)
    for w in range(NW + 1):
        lo = jnp.int32(-1) if w == 0 else pid_ref[16 * (w - 1) + 1]
        hi = jnp.int32(NUM_SEGMENTS) if w == NW else pid_ref[16 * w]
        clear = jnp.logical_or(clear, jnp.logical_and(sid > lo, sid < hi))
    for k in range(2 * NW):
        pid = pid_ref[16 * (k // 2) + (k % 2)]
        clear = jnp.logical_or(clear, sid == pid)

    val = jnp.where(clear, 0.0, d_ref[...])
    for k in range(2 * NW):
        pid = pid_ref[16 * (k // 2) + (k % 2)]
        prow = p_ref[k:k + 1, :]
        val = jnp.where(sid == pid, jnp.maximum(val, prow), val)

    o_ref[...] = (jnp.dot(val.astype(jnp.bfloat16), w3_ref[...],
                          preferred_element_type=jnp.float32)
                  + b3_ref[...])


def _final(direct, partials, pids, W3, b3):
    grid = (NUM_SEGMENTS // SEG_TILE,)
    return pl.pallas_call(
        _final_body,
        grid=grid,
        in_specs=[
            pl.BlockSpec((SEG_TILE, LATENT), lambda i: (i, 0)),
            pl.BlockSpec((2 * NW, LATENT), lambda i: (0, 0)),
            pl.BlockSpec(memory_space=pltpu.SMEM),
            pl.BlockSpec((LATENT, LATENT), lambda i: (0, 0)),
            pl.BlockSpec((1, LATENT), lambda i: (0, 0)),
        ],
        out_specs=pl.BlockSpec((SEG_TILE, LATENT), lambda i: (i, 0)),
        out_shape=jax.ShapeDtypeStruct((NUM_SEGMENTS, LATENT), jnp.float32),
    )(direct, partials, pids, W3, b3)


def kernel(feat, traj_inbatch_index, W1, b1, W2, b2, W3, b3):
    idx = traj_inbatch_index.astype(jnp.int32)
    emb = _mlp(feat.astype(jnp.bfloat16), W1.astype(jnp.bfloat16),
               b1.reshape(1, HIDDEN), W2.astype(jnp.bfloat16),
               b2.reshape(1, LATENT))
    direct, partials, pids = _segmax(emb, idx)
    return _final(direct.reshape(NUM_SEGMENTS, LATENT),
                  partials.reshape(2 * NW, LATENT), pids,
                  W3.astype(jnp.bfloat16), b3.reshape(1, LATENT))


# speedup vs baseline: 1.6342x; 1.1295x over previous
import jax
import jax.numpy as jnp
from jax import lax
from jax.experimental import pallas as pl
from jax.experimental.pallas import tpu as pltpu
from jax.experimental.pallas import tpu_sc as plsc

N = 320000
FEAT_DIM = 128
HIDDEN = 512
LATENT = 128
NUM_SEGMENTS = 10000
NJ = LATENT // 16

NC = 2
NS = 16
NW = NC * NS
ROWS_PER_W = N // NW
CHUNK = 400
NCHUNK = ROWS_PER_W // CHUNK
NBLK = CHUNK // 16
RING = 8

ROW_TILE = 800
SEG_TILE = 1000


def _mlp_body(x_ref, w1_ref, b1_ref, w2_ref, b2_ref, o_ref):
    h = jnp.maximum(
        jnp.dot(x_ref[...].astype(jnp.bfloat16), w1_ref[...],
                preferred_element_type=jnp.float32)
        + b1_ref[...], 0.0)
    e = jnp.maximum(
        jnp.dot(h.astype(jnp.bfloat16), w2_ref[...],
                preferred_element_type=jnp.float32)
        + b2_ref[...], 0.0)
    o_ref[...] = e


def _mlp(feat, W1, b1, W2, b2):
    grid = (N // ROW_TILE,)
    return pl.pallas_call(
        _mlp_body,
        grid=grid,
        in_specs=[
            pl.BlockSpec((ROW_TILE, FEAT_DIM), lambda i: (i, 0)),
            pl.BlockSpec((FEAT_DIM, HIDDEN), lambda i: (0, 0)),
            pl.BlockSpec((1, HIDDEN), lambda i: (0, 0)),
            pl.BlockSpec((HIDDEN, LATENT), lambda i: (0, 0)),
            pl.BlockSpec((1, LATENT), lambda i: (0, 0)),
        ],
        out_specs=pl.BlockSpec((ROW_TILE, LATENT), lambda i: (i, 0)),
        out_shape=jax.ShapeDtypeStruct((N, LATENT), jnp.float32),
    )(feat, W1, b1, W2, b2)


def _segmax_body(emb_hbm, idx_hbm, direct_hbm, partials_hbm, pids_hbm,
                 idx_v, buf_v, mbuf_v, stage_v, zrow_v, pid_v,
                 flush_sem, chunk_sem):
    w = lax.axis_index("s") * NC + lax.axis_index("c")
    base = w * ROWS_PER_W
    zero16i = jnp.zeros((16,), jnp.int32)
    zero16f = jnp.zeros((16,), jnp.float32)

    idx_v[pl.ds(0, 16)] = zero16i - 1
    pltpu.sync_copy(idx_hbm.at[pl.ds(base, ROWS_PER_W)],
                    idx_v.at[pl.ds(16, ROWS_PER_W)])

    def idx_at(r):
        return idx_v[pl.ds(16 + r, 16)][0]

    for j in range(NJ):
        zrow_v[pl.ds(j * 16, 16)] = zero16f
        mbuf_v[pl.ds(j * 16, 16)] = zero16f

    def chunk_src(c):
        return emb_hbm.at[pl.ds((base + c * CHUNK) * LATENT, CHUNK * LATENT)]

    def chunk_dst(c):
        return buf_v.at[pl.ds(lax.rem(c, 2) * (CHUNK * LATENT),
                              CHUNK * LATENT)]

    def start_chunk(c):
        pltpu.async_copy(chunk_src(c), chunk_dst(c),
                         chunk_sem.at[lax.rem(c, 2)])

    def wait_chunk(c):
        pltpu.make_async_copy(chunk_src(c), chunk_dst(c),
                              chunk_sem.at[lax.rem(c, 2)]).wait()

    def stage_slot(slot):
        return stage_v.at[pl.ds(slot * LATENT, LATENT)]

    def flush(cur, m, first_open, k):
        slot = lax.rem(k, RING)

        def wait_slot():
            pltpu.make_async_copy(stage_slot(slot),
                                  direct_hbm.at[pl.ds(0, LATENT)],
                                  flush_sem.at[slot]).wait()

        lax.cond(jnp.logical_and(first_open == 0, k >= RING),
                 wait_slot, lambda: None)
        for j in range(NJ):
            stage_v[pl.ds(slot * LATENT + j * 16, 16)] = m[j]

        def to_partial():
            pltpu.sync_copy(stage_slot(slot),
                            partials_hbm.at[pl.ds(2 * w * LATENT, LATENT)])

        def to_direct():
            pltpu.async_copy(stage_slot(slot),
                             direct_hbm.at[pl.ds(cur * LATENT, LATENT)],
                             flush_sem.at[slot])

        lax.cond(first_open == 1, to_partial, to_direct)
        return jnp.where(first_open == 1, k, k + 1)

    def zero_gap(lo, hi):
        def body(g, _):
            pltpu.sync_copy(zrow_v, direct_hbm.at[pl.ds(g * LATENT, LATENT)])
            return 0
        lax.fori_loop(lo, hi, body, 0)

    def load_m():
        return tuple(mbuf_v[pl.ds(j * 16, 16)] for j in range(NJ))

    def store_m(m):
        for j in range(NJ):
            mbuf_v[pl.ds(j * 16, 16)] = m[j]

    def row_vals(boff, r):
        return tuple(buf_v[pl.ds(boff + r * LATENT + j * 16, 16)]
                     for j in range(NJ))

    def block_body(b, carry):
        c, cur, first_open, k = carry
        g = b * 16
        boff = (lax.rem(c, 2) * CHUNK + (g - c * CHUNK)) * LATENT
        a_vec = idx_v[pl.ds(16 + g, 16)]
        p_vec = idx_v[pl.ds(15 + g, 16)]
        nb = plsc.all_reduce_population_count(a_vec != p_vec)[0]

        def fast():
            m = load_m()
            rows = [row_vals(boff, r) for r in range(16)]
            while len(rows) > 1:
                rows = [tuple(jnp.maximum(x[j], y[j]) for j in range(NJ))
                        for x, y in zip(rows[::2], rows[1::2])]
            store_m(tuple(jnp.maximum(m[j], rows[0][j]) for j in range(NJ)))
            return cur, first_open, k

        def slow():
            def row_body(r, rcarry):
                rcur, ropen, rk = rcarry
                s = idx_at(g + r)
                v = row_vals(boff, r)
                changed = s != rcur
                m = load_m()

                def on_change(_):
                    nk = flush(rcur, m, ropen, rk)
                    zero_gap(rcur + 1, s)
                    return nk

                nk = lax.cond(changed, on_change, lambda _: rk, 0)
                store_m(tuple(
                    jnp.where(changed, v[j], jnp.maximum(m[j], v[j]))
                    for j in range(NJ)))
                return (jnp.where(changed, s, rcur),
                        jnp.where(changed, jnp.int32(0), ropen),
                        nk)

            return lax.fori_loop(0, 16, row_body, (cur, first_open, k))

        cur2, open2, k2 = lax.cond(nb == 0, fast, slow)
        return c, cur2, open2, k2

    def chunk_body(c, carry):
        def prefetch():
            start_chunk(c + 1)
        lax.cond(c < NCHUNK - 1, prefetch, lambda: None)
        wait_chunk(c)
        cur, first_open, k = carry
        _, cur, first_open, k = lax.fori_loop(
            c * NBLK, (c + 1) * NBLK, block_body, (c, cur, first_open, k))
        return cur, first_open, k

    start_chunk(0)
    init = (idx_at(0), jnp.int32(1), jnp.int32(0))
    cur, first_open, k = lax.fori_loop(0, NCHUNK, chunk_body, init)

    for s in range(RING):
        def drain():
            pltpu.make_async_copy(stage_slot(s),
                                  direct_hbm.at[pl.ds(0, LATENT)],
                                  flush_sem.at[s]).wait()
        lax.cond(k > s, drain, lambda: None)

    m = load_m()
    for j in range(NJ):
        stage_v[pl.ds(j * 16, 16)] = m[j]
    pltpu.sync_copy(stage_v.at[pl.ds(0, LATENT)],
                    partials_hbm.at[pl.ds((2 * w + 1) * LATENT, LATENT)])

    def also_first():
        pltpu.sync_copy(stage_v.at[pl.ds(0, LATENT)],
                        partials_hbm.at[pl.ds(2 * w * LATENT, LATENT)])

    lax.cond(first_open == 1, also_first, lambda: None)

    lane = lax.broadcasted_iota(jnp.int32, (16,), 0)
    pid_v[...] = jnp.where(lane == 0, idx_at(0),
                           jnp.where(lane == 1, cur, 0))
    pltpu.sync_copy(pid_v, pids_hbm.at[pl.ds(w * 16, 16)])


def _segmax(emb, idx):
    mesh = plsc.VectorSubcoreMesh(core_axis_name="c", subcore_axis_name="s")
    f = pl.kernel(
        _segmax_body,
        out_type=(
            jax.ShapeDtypeStruct((NUM_SEGMENTS * LATENT,), jnp.float32),
            jax.ShapeDtypeStruct((2 * NW * LATENT,), jnp.float32),
            jax.ShapeDtypeStruct((NW * 16,), jnp.int32),
        ),
        mesh=mesh,
        compiler_params=pltpu.CompilerParams(use_tc_tiling_on_sc=False,
                                             needs_layout_passes=False),
        scratch_types=[
            pltpu.VMEM((32 + ROWS_PER_W + 16,), jnp.int32),
            pltpu.VMEM((2 * CHUNK * LATENT,), jnp.float32),
            pltpu.VMEM((LATENT,), jnp.float32),
            pltpu.VMEM((RING * LATENT,), jnp.float32),
            pltpu.VMEM((LATENT,), jnp.float32),
            pltpu.VMEM((16,), jnp.int32),
            pltpu.SemaphoreType.DMA((RING,)),
            pltpu.SemaphoreType.DMA((2,)),
        ],
    )
    return f(emb.reshape(-1), idx)


def _final_body(d_ref, p_ref, pid_ref, w3_ref, b3_ref, o_ref,
                pm_ref, val_ref):
    i = pl.program_id(0)
    sid = lax.broadcasted_iota(jnp.int32, (SEG_TILE, 1), 0) + i * SEG_TILE

    @pl.when(i == 0)
    def _():
        krow = lax.broadcasted_iota(jnp.int32, (2 * NW, 1), 0)
        pids_col = jnp.zeros((2 * NW, 1), jnp.int32)
        for k in range(2 * NW):
            pid = pid_ref[16 * (k // 2) + (k % 2)]
            pids_col = jnp.where(krow == k, pid, pids_col)
        p = p_ref[...]
        pm = p
        for k in range(2 * NW):
            pid = pid_ref[16 * (k // 2) + (k % 2)]
            m = jnp.max(jnp.where(pids_col == pid, p, 0.0),
                        axis=0, keepdims=True)
            pm = jnp.where(krow == k, m, pm)
        pm_ref[...] = pm

    clear = jnp.zeros((SEG_TILE, 1), jnp.bool_)
    for w in range(NW + 1):
        lo = jnp.int32(-1) if w == 0 else pid_ref[16 * (w - 1) + 1]
        hi = jnp.int32(NUM_SEGMENTS) if w == NW else pid_ref[16 * w]
        clear = jnp.logical_or(clear, jnp.logical_and(sid > lo, sid < hi))

    val_ref[...] = jnp.where(clear, 0.0, d_ref[...])

    for k in range(2 * NW):
        pid = pid_ref[16 * (k // 2) + (k % 2)]

        @pl.when(jnp.logical_and(pid >= i * SEG_TILE,
                                 pid < (i + 1) * SEG_TILE))
        def _():
            val_ref[pl.ds(pid - i * SEG_TILE, 1), :] = pm_ref[k:k + 1, :]

    o_ref[...] = (jnp.dot(val_ref[...].astype(jnp.bfloat16), w3_ref[...],
                          preferred_element_type=jnp.float32)
                  + b3_ref[...])


def _final(direct, partials, pids, W3, b3):
    grid = (NUM_SEGMENTS // SEG_TILE,)
    return pl.pallas_call(
        _final_body,
        grid=grid,
        in_specs=[
            pl.BlockSpec((SEG_TILE, LATENT), lambda i: (i, 0)),
            pl.BlockSpec((2 * NW, LATENT), lambda i: (0, 0)),
            pl.BlockSpec(memory_space=pltpu.SMEM),
            pl.BlockSpec((LATENT, LATENT), lambda i: (0, 0)),
            pl.BlockSpec((1, LATENT), lambda i: (0, 0)),
        ],
        out_specs=pl.BlockSpec((SEG_TILE, LATENT), lambda i: (i, 0)),
        out_shape=jax.ShapeDtypeStruct((NUM_SEGMENTS, LATENT), jnp.float32),
        scratch_shapes=[
            pltpu.VMEM((2 * NW, LATENT), jnp.float32),
            pltpu.VMEM((SEG_TILE, LATENT), jnp.float32),
        ],
    )(direct, partials, pids, W3, b3)


def kernel(feat, traj_inbatch_index, W1, b1, W2, b2, W3, b3):
    idx = traj_inbatch_index.astype(jnp.int32)
    emb = _mlp(feat.astype(jnp.bfloat16), W1.astype(jnp.bfloat16),
               b1.reshape(1, HIDDEN), W2.astype(jnp.bfloat16),
               b2.reshape(1, LATENT))
    direct, partials, pids = _segmax(emb, idx)
    return _final(direct.reshape(NUM_SEGMENTS, LATENT),
                  partials.reshape(2 * NW, LATENT), pids,
                  W3.astype(jnp.bfloat16), b3.reshape(1, LATENT))


# speedup vs baseline: 1.7450x; 1.0678x over previous
import jax
import jax.numpy as jnp
from jax import lax
from jax.experimental import pallas as pl
from jax.experimental.pallas import tpu as pltpu
from jax.experimental.pallas import tpu_sc as plsc

N = 320000
FEAT_DIM = 128
HIDDEN = 512
LATENT = 128
NUM_SEGMENTS = 10000
NJ = LATENT // 16

NC = 2
NS = 16
NW = NC * NS
ROWS_PER_W = N // NW
CHUNK = 400
NCHUNK = ROWS_PER_W // CHUNK
NBLK = CHUNK // 16
RING = 8

ROW_TILE = 800
SEG_TILE = 1000


def _mlp_body(x_ref, w1_ref, b1_ref, w2_ref, b2_ref, o_ref):
    h = jnp.maximum(
        jnp.dot(x_ref[...].astype(jnp.bfloat16), w1_ref[...],
                preferred_element_type=jnp.float32)
        + b1_ref[...], 0.0)
    e = jnp.maximum(
        jnp.dot(h.astype(jnp.bfloat16), w2_ref[...],
                preferred_element_type=jnp.float32)
        + b2_ref[...], 0.0)
    o_ref[...] = e


def _mlp(feat, W1, b1, W2, b2):
    grid = (N // ROW_TILE,)
    return pl.pallas_call(
        _mlp_body,
        grid=grid,
        in_specs=[
            pl.BlockSpec((ROW_TILE, FEAT_DIM), lambda i: (i, 0)),
            pl.BlockSpec((FEAT_DIM, HIDDEN), lambda i: (0, 0)),
            pl.BlockSpec((1, HIDDEN), lambda i: (0, 0)),
            pl.BlockSpec((HIDDEN, LATENT), lambda i: (0, 0)),
            pl.BlockSpec((1, LATENT), lambda i: (0, 0)),
        ],
        out_specs=pl.BlockSpec((ROW_TILE, LATENT), lambda i: (i, 0)),
        out_shape=jax.ShapeDtypeStruct((N, LATENT), jnp.float32),
    )(feat, W1, b1, W2, b2)


def _segmax_body(emb_hbm, idx_hbm, direct_hbm, partials_hbm, pids_hbm,
                 idx_v, buf_v, mbuf_v, stage_v, zrow_v, pid_v,
                 flush_sem, chunk_sem):
    w = lax.axis_index("s") * NC + lax.axis_index("c")
    base = w * ROWS_PER_W
    zero16i = jnp.zeros((16,), jnp.int32)
    zero16f = jnp.zeros((16,), jnp.float32)

    idx_v[pl.ds(0, 16)] = zero16i - 1
    pltpu.sync_copy(idx_hbm.at[pl.ds(base, ROWS_PER_W)],
                    idx_v.at[pl.ds(16, ROWS_PER_W)])

    def idx_at(r):
        return idx_v[pl.ds(16 + r, 16)][0]

    for j in range(NJ):
        zrow_v[pl.ds(j * 16, 16)] = zero16f
        mbuf_v[pl.ds(j * 16, 16)] = zero16f

    def chunk_src(c):
        return emb_hbm.at[pl.ds((base + c * CHUNK) * LATENT, CHUNK * LATENT)]

    def chunk_dst(c):
        return buf_v.at[pl.ds(lax.rem(c, 2) * (CHUNK * LATENT),
                              CHUNK * LATENT)]

    def start_chunk(c):
        pltpu.async_copy(chunk_src(c), chunk_dst(c),
                         chunk_sem.at[lax.rem(c, 2)])

    def wait_chunk(c):
        pltpu.make_async_copy(chunk_src(c), chunk_dst(c),
                              chunk_sem.at[lax.rem(c, 2)]).wait()

    def stage_slot(slot):
        return stage_v.at[pl.ds(slot * LATENT, LATENT)]

    def flush(cur, m, first_open, k):
        slot = lax.rem(k, RING)

        def wait_slot():
            pltpu.make_async_copy(stage_slot(slot),
                                  direct_hbm.at[pl.ds(0, LATENT)],
                                  flush_sem.at[slot]).wait()

        lax.cond(jnp.logical_and(first_open == 0, k >= RING),
                 wait_slot, lambda: None)
        for j in range(NJ):
            stage_v[pl.ds(slot * LATENT + j * 16, 16)] = m[j]

        def to_partial():
            pltpu.sync_copy(stage_slot(slot),
                            partials_hbm.at[pl.ds(2 * w * LATENT, LATENT)])

        def to_direct():
            pltpu.async_copy(stage_slot(slot),
                             direct_hbm.at[pl.ds(cur * LATENT, LATENT)],
                             flush_sem.at[slot])

        lax.cond(first_open == 1, to_partial, to_direct)
        return jnp.where(first_open == 1, k, k + 1)

    def zero_gap(lo, hi):
        def body(g, _):
            pltpu.sync_copy(zrow_v, direct_hbm.at[pl.ds(g * LATENT, LATENT)])
            return 0
        lax.fori_loop(lo, hi, body, 0)

    def load_m():
        return tuple(mbuf_v[pl.ds(j * 16, 16)] for j in range(NJ))

    def store_m(m):
        for j in range(NJ):
            mbuf_v[pl.ds(j * 16, 16)] = m[j]

    def row_vals(boff, r):
        return tuple(buf_v[pl.ds(boff + r * LATENT + j * 16, 16)]
                     for j in range(NJ))

    def block_body(b, carry):
        c, cur, first_open, k = carry
        g = b * 16
        boff = (lax.rem(c, 2) * CHUNK + (g - c * CHUNK)) * LATENT
        a_vec = idx_v[pl.ds(16 + g, 16)]
        p_vec = idx_v[pl.ds(15 + g, 16)]
        nb = plsc.all_reduce_population_count(a_vec != p_vec)[0]

        def fast():
            m = load_m()
            rows = [row_vals(boff, r) for r in range(16)]
            while len(rows) > 1:
                rows = [tuple(jnp.maximum(x[j], y[j]) for j in range(NJ))
                        for x, y in zip(rows[::2], rows[1::2])]
            store_m(tuple(jnp.maximum(m[j], rows[0][j]) for j in range(NJ)))
            return cur, first_open, k

        def slow():
            def row_body(r, rcarry):
                rcur, ropen, rk = rcarry
                s = idx_at(g + r)
                v = row_vals(boff, r)
                changed = s != rcur
                m = load_m()

                def on_change(_):
                    nk = flush(rcur, m, ropen, rk)
                    zero_gap(rcur + 1, s)
                    return nk

                nk = lax.cond(changed, on_change, lambda _: rk, 0)
                store_m(tuple(
                    jnp.where(changed, v[j], jnp.maximum(m[j], v[j]))
                    for j in range(NJ)))
                return (jnp.where(changed, s, rcur),
                        jnp.where(changed, jnp.int32(0), ropen),
                        nk)

            return lax.fori_loop(0, 16, row_body, (cur, first_open, k))

        cur2, open2, k2 = lax.cond(nb == 0, fast, slow)
        return c, cur2, open2, k2

    def chunk_body(c, carry):
        def prefetch():
            start_chunk(c + 1)
        lax.cond(c < NCHUNK - 1, prefetch, lambda: None)
        wait_chunk(c)
        cur, first_open, k = carry
        _, cur, first_open, k = lax.fori_loop(
            c * NBLK, (c + 1) * NBLK, block_body, (c, cur, first_open, k))
        return cur, first_open, k

    start_chunk(0)
    init = (idx_at(0), jnp.int32(1), jnp.int32(0))
    cur, first_open, k = lax.fori_loop(0, NCHUNK, chunk_body, init)

    for s in range(RING):
        def drain():
            pltpu.make_async_copy(stage_slot(s),
                                  direct_hbm.at[pl.ds(0, LATENT)],
                                  flush_sem.at[s]).wait()
        lax.cond(k > s, drain, lambda: None)

    m = load_m()
    for j in range(NJ):
        stage_v[pl.ds(j * 16, 16)] = m[j]
    pltpu.sync_copy(stage_v.at[pl.ds(0, LATENT)],
                    partials_hbm.at[pl.ds((2 * w + 1) * LATENT, LATENT)])

    def also_first():
        pltpu.sync_copy(stage_v.at[pl.ds(0, LATENT)],
                        partials_hbm.at[pl.ds(2 * w * LATENT, LATENT)])

    lax.cond(first_open == 1, also_first, lambda: None)

    lane = lax.broadcasted_iota(jnp.int32, (16,), 0)
    pid_v[...] = jnp.where(lane == 0, idx_at(0),
                           jnp.where(lane == 1, cur, 0))
    pltpu.sync_copy(pid_v, pids_hbm.at[pl.ds(w * 16, 16)])


def _segmax(emb, idx):
    mesh = plsc.VectorSubcoreMesh(core_axis_name="c", subcore_axis_name="s")
    f = pl.kernel(
        _segmax_body,
        out_type=(
            jax.ShapeDtypeStruct((NUM_SEGMENTS * LATENT,), jnp.float32),
            jax.ShapeDtypeStruct((2 * NW * LATENT,), jnp.float32),
            jax.ShapeDtypeStruct((NW * 16,), jnp.int32),
        ),
        mesh=mesh,
        compiler_params=pltpu.CompilerParams(use_tc_tiling_on_sc=False,
                                             needs_layout_passes=False),
        scratch_types=[
            pltpu.VMEM((32 + ROWS_PER_W + 16,), jnp.int32),
            pltpu.VMEM((2 * CHUNK * LATENT,), jnp.float32),
            pltpu.VMEM((LATENT,), jnp.float32),
            pltpu.VMEM((RING * LATENT,), jnp.float32),
            pltpu.VMEM((LATENT,), jnp.float32),
            pltpu.VMEM((16,), jnp.int32),
            pltpu.SemaphoreType.DMA((RING,)),
            pltpu.SemaphoreType.DMA((2,)),
        ],
    )
    return f(emb.reshape(-1), idx)


def _final_body(d_ref, p_ref, pid_ref, w3_ref, b3_ref, o_ref,
                pm_ref, val_ref):
    i = pl.program_id(0)
    sid = lax.broadcasted_iota(jnp.int32, (SEG_TILE, 1), 0) + i * SEG_TILE

    @pl.when(i == 0)
    def _():
        krow = lax.broadcasted_iota(jnp.int32, (2 * NW, 1), 0)
        pids_col = jnp.zeros((2 * NW, 1), jnp.int32)
        for k in range(2 * NW):
            pid = pid_ref[16 * (k // 2) + (k % 2)]
            pids_col = jnp.where(krow == k, pid, pids_col)
        p = p_ref[...]
        pm = p
        for k in range(2 * NW):
            pid = pid_ref[16 * (k // 2) + (k % 2)]
            m = jnp.max(jnp.where(pids_col == pid, p, 0.0),
                        axis=0, keepdims=True)
            pm = jnp.where(krow == k, m, pm)
        pm_ref[...] = pm

    clear = jnp.zeros((SEG_TILE, 1), jnp.bool_)
    for w in range(NW + 1):
        lo = jnp.int32(-1) if w == 0 else pid_ref[16 * (w - 1) + 1]
        hi = jnp.int32(NUM_SEGMENTS) if w == NW else pid_ref[16 * w]
        clear = jnp.logical_or(clear, jnp.logical_and(sid > lo, sid < hi))

    val_ref[...] = jnp.where(clear, 0.0, d_ref[...])

    for k in range(2 * NW):
        pid = pid_ref[16 * (k // 2) + (k % 2)]

        @pl.when(jnp.logical_and(pid >= i * SEG_TILE,
                                 pid < (i + 1) * SEG_TILE))
        def _():
            val_ref[pl.ds(pid - i * SEG_TILE, 1), :] = pm_ref[k:k + 1, :]

    o_ref[...] = (jnp.dot(val_ref[...].astype(jnp.bfloat16), w3_ref[...],
                          preferred_element_type=jnp.float32)
                  + b3_ref[...])


def _final(direct, partials, pids, W3, b3):
    grid = (NUM_SEGMENTS // SEG_TILE,)
    return pl.pallas_call(
        _final_body,
        grid=grid,
        in_specs=[
            pl.BlockSpec((SEG_TILE, LATENT), lambda i: (i, 0)),
            pl.BlockSpec((2 * NW, LATENT), lambda i: (0, 0)),
            pl.BlockSpec(memory_space=pltpu.SMEM),
            pl.BlockSpec((LATENT, LATENT), lambda i: (0, 0)),
            pl.BlockSpec((1, LATENT), lambda i: (0, 0)),
        ],
        out_specs=pl.BlockSpec((SEG_TILE, LATENT), lambda i: (i, 0)),
        out_shape=jax.ShapeDtypeStruct((NUM_SEGMENTS, LATENT), jnp.float32),
        scratch_shapes=[
            pltpu.VMEM((2 * NW, LATENT), jnp.float32),
            pltpu.VMEM((SEG_TILE, LATENT), jnp.float32),
        ],
    )(direct, partials, pids, W3, b3)


def kernel(feat, traj_inbatch_index, W1, b1, W2, b2, W3, b3):
    idx = traj_inbatch_index.astype(jnp.int32)
    emb = _mlp(feat, W1.astype(jnp.bfloat16),
               b1.reshape(1, HIDDEN), W2.astype(jnp.bfloat16),
               b2.reshape(1, LATENT))
    direct, partials, pids = _segmax(emb, idx)
    return _final(direct.reshape(NUM_SEGMENTS, LATENT),
                  partials.reshape(2 * NW, LATENT), pids,
                  W3.astype(jnp.bfloat16), b3.reshape(1, LATENT))
